# Initial kernel scaffold; baseline (speedup 1.0000x reference)
import jax
import jax.numpy as jnp
from jax.experimental import pallas as pl


def kernel(feature, edge_index, edge_type, norm, weight):
    raise NotImplementedError("write your pallas kernel here")


# speedup vs baseline: 18.8337x; 18.8337x over previous
import functools

import jax
import jax.numpy as jnp
from jax import lax
from jax.experimental import pallas as pl
from jax.experimental.pallas import tpu as pltpu
from jax.experimental.pallas import tpu_sc as plsc

N = 10000
E = 320000
D = 128
R = 8

NC = 2
NS = 16
NW = NC * NS
EPT = E // NW
CHUNK = 125
NCHUNK = EPT // CHUNK
ROWS_PT = 624
TAIL_ROWS = N - 16 * ROWS_PT
ZBLK = 104
DH = D // 2
LANES_H = DH // 16


def _mm_body(f_ref, w_ref, o_ref):
    h = jnp.dot(f_ref[...], w_ref[0], preferred_element_type=jnp.float32)
    o_ref[0, 0] = h[:, :DH]
    o_ref[1, 0] = h[:, DH:]


def _relation_transform(feature, weight):
    bn = 2000
    return pl.pallas_call(
        _mm_body,
        grid=(N // bn, R),
        in_specs=[
            pl.BlockSpec((bn, D), lambda b, r: (b, 0)),
            pl.BlockSpec((1, D, D), lambda b, r: (r, 0, 0)),
        ],
        out_specs=pl.BlockSpec((2, 1, bn, DH), lambda b, r: (0, r, b, 0)),
        out_shape=jax.ShapeDtypeStruct((2, R, N, DH), jnp.float32),
    )(feature, weight)


def _add_body(p_ref, o_ref):
    o_ref[...] = jnp.concatenate(
        [p_ref[0, 0] + p_ref[1, 0], p_ref[0, 1] + p_ref[1, 1]], axis=-1
    )


def _combine_partials(partials):
    bn = 2000
    return pl.pallas_call(
        _add_body,
        grid=(N // bn,),
        in_specs=[pl.BlockSpec((NC, 2, bn, DH), lambda b: (0, 0, b, 0))],
        out_specs=pl.BlockSpec((bn, D), lambda b: (b, 0)),
        out_shape=jax.ShapeDtypeStruct((N, D), jnp.float32),
    )(partials)


def _sc_body(hr_hbm, g_hbm, dst_hbm, norm_hbm, out_hbm,
             g_v, d_v, n_v, gbuf0, gbuf1, sbuf0, sbuf1, acc,
             gsem0, gsem1, ssem0, ssem1):
    cid = lax.axis_index("c")
    sid = lax.axis_index("s")
    w = cid * NS + sid

    gbufs = (gbuf0, gbuf1)
    sbufs = (sbuf0, sbuf1)
    gsems = (gsem0, gsem1)
    ssems = (ssem0, ssem1)

    pltpu.sync_copy(g_hbm.at[w], g_v)
    pltpu.sync_copy(dst_hbm.at[w], d_v)
    pltpu.sync_copy(norm_hbm.at[w], n_v)

    zeros16 = jnp.zeros((16,), jnp.float32)

    for h in range(2):
        hr_h = hr_hbm.at[h]

        @pl.loop(0, ZBLK)
        def _(e):
            for k in range(LANES_H):
                gbuf0[e, pl.ds(k * 16, 16)] = zeros16

        for j in range(ROWS_PT // ZBLK):
            pltpu.sync_copy(gbuf0.at[pl.ds(0, ZBLK)],
                            acc.at[pl.ds(sid * ROWS_PT + j * ZBLK, ZBLK)])

        @pl.when(sid == NS - 1)
        def _():
            pltpu.sync_copy(gbuf0.at[pl.ds(0, TAIL_ROWS)],
                            acc.at[pl.ds(N - TAIL_ROWS, TAIL_ROWS)])

        def start_gather(c, b):
            pltpu.async_copy(hr_h.at[g_v.at[c]], gbufs[b], gsems[b])

        def wait_gather(c, b):
            pltpu.make_async_copy(hr_h.at[g_v.at[c]], gbufs[b], gsems[b]).wait()

        def start_scatter(c, b):
            pltpu.async_copy(sbufs[b], acc.at[d_v.at[c]], ssems[b], add=True)

        def wait_scatter(c, b):
            pltpu.make_async_copy(sbufs[b], acc.at[d_v.at[c]], ssems[b]).wait()

        plsc.subcore_barrier()

        start_gather(0, 0)
        start_gather(1, 1)

        @pl.loop(0, NCHUNK, step=2)
        def _(c0):
            for b in range(2):
                c = c0 + b
                wait_gather(c, b)

                @pl.when(c0 > 0)
                def _():
                    wait_scatter(c - 2, b)

                def scale_group(off):
                    nv = n_v[c, pl.ds(off, 16)]
                    for i in range(16):
                        s = nv[i]
                        for k in range(LANES_H):
                            sl = pl.ds(k * 16, 16)
                            sbufs[b][off + i, sl] = gbufs[b][off + i, sl] * s

                @pl.loop(0, CHUNK // 16)
                def _(eg):
                    scale_group(eg * 16)

                scale_group(CHUNK - 16)

                @pl.when(c0 < NCHUNK - 2)
                def _():
                    start_gather(c + 2, b)

                start_scatter(c, b)

        wait_scatter(NCHUNK - 2, 0)
        wait_scatter(NCHUNK - 1, 1)

        plsc.subcore_barrier()
        pltpu.sync_copy(acc.at[pl.ds(sid * ROWS_PT, ROWS_PT)],
                        out_hbm.at[cid, h, pl.ds(sid * ROWS_PT, ROWS_PT)])

        @pl.when(sid == NS - 1)
        def _():
            pltpu.sync_copy(acc.at[pl.ds(N - TAIL_ROWS, TAIL_ROWS)],
                            out_hbm.at[cid, h, pl.ds(N - TAIL_ROWS, TAIL_ROWS)])


@functools.partial(
    pl.kernel,
    out_type=jax.ShapeDtypeStruct((NC, 2, N, DH), jnp.float32),
    mesh=plsc.VectorSubcoreMesh(
        core_axis_name="c", subcore_axis_name="s", num_cores=NC, num_subcores=NS
    ),
    compiler_params=pltpu.CompilerParams(use_tc_tiling_on_sc=False),
    scratch_types=[
        pltpu.VMEM((NCHUNK, CHUNK), jnp.int32),
        pltpu.VMEM((NCHUNK, CHUNK), jnp.int32),
        pltpu.VMEM((NCHUNK, CHUNK), jnp.float32),
        pltpu.VMEM((CHUNK, DH), jnp.float32),
        pltpu.VMEM((CHUNK, DH), jnp.float32),
        pltpu.VMEM((CHUNK, DH), jnp.float32),
        pltpu.VMEM((CHUNK, DH), jnp.float32),
        pltpu.VMEM_SHARED((N, DH), jnp.float32),
        pltpu.SemaphoreType.DMA,
        pltpu.SemaphoreType.DMA,
        pltpu.SemaphoreType.DMA,
        pltpu.SemaphoreType.DMA,
    ],
)
def _sc_edge_kernel(hr_hbm, g_hbm, dst_hbm, norm_hbm, out_hbm, *rest):
    _sc_body(hr_hbm, g_hbm, dst_hbm, norm_hbm, out_hbm, *rest)


def kernel(feature, edge_index, edge_type, norm, weight):
    hr = _relation_transform(feature, weight).reshape(2, R * N, DH)
    src = edge_index[0]
    dst = edge_index[1]
    g = (edge_type.astype(jnp.int32) * N + src).reshape(NW, NCHUNK, CHUNK)
    dst3 = dst.reshape(NW, NCHUNK, CHUNK)
    norm3 = norm.reshape(NW, NCHUNK, CHUNK)
    partials = _sc_edge_kernel(hr, g, dst3, norm3)
    return _combine_partials(partials)
